# Initial kernel scaffold; baseline (speedup 1.0000x reference)
import jax
import jax.numpy as jnp
from jax.experimental import pallas as pl


def kernel(output, target, train_mask, edge_index, x):
    raise NotImplementedError("write your pallas kernel here")


# speedup vs baseline: 4.7475x; 4.7475x over previous
import jax
import jax.numpy as jnp
from jax import lax
from jax.experimental import pallas as pl
from jax.experimental.pallas import tpu as pltpu
from jax.experimental.pallas import tpu_sc as plsc

N = 10000
C = 128
E = 320000
MU = 0.01

NC, NS, L = 2, 16, 16
NW = NC * NS
K = 128
NCHUNK = -(-E // K)
_CPW_RAW = -(-NCHUNK // NW)
CPW = ((_CPW_RAW + 7) // 8) * 8
NCH_PAD = CPW * NW
EP = NCH_PAD * K
NPAD = 10240
NSLICE = NPAD // NS


def _degree_body(row_hbm, deg_out, idx_all, ones_v, slice_v, deg_sh):
    c = lax.axis_index("c")
    s = lax.axis_index("s")
    w = s * NC + c

    def zb(k, carry):
        slice_v[pl.ds(k * L, L)] = jnp.zeros((L,), jnp.float32)
        return carry

    lax.fori_loop(0, NSLICE // L, zb, 0)
    pltpu.sync_copy(slice_v, deg_sh.at[pl.ds(s * NSLICE, NSLICE)])
    for t in range(K // L):
        ones_v[pl.ds(t * L, L)] = jnp.ones((L,), jnp.float32)
    pltpu.sync_copy(row_hbm.at[pl.ds(w * CPW, CPW)], idx_all)
    plsc.subcore_barrier()

    def body(j, carry):
        pltpu.sync_copy(ones_v, deg_sh.at[idx_all.at[j]], add=True)
        return carry

    lax.fori_loop(0, CPW, body, 0)
    plsc.subcore_barrier()
    pltpu.sync_copy(deg_sh.at[pl.ds(s * NSLICE, NSLICE)], slice_v)
    pltpu.sync_copy(slice_v, deg_out.at[c, pl.ds(s * NSLICE, NSLICE)])


def _degree_call(row2d):
    return pl.kernel(
        _degree_body,
        out_type=jax.ShapeDtypeStruct((NC, NPAD), jnp.float32),
        mesh=plsc.VectorSubcoreMesh(core_axis_name="c", subcore_axis_name="s"),
        scratch_types=[
            pltpu.VMEM((CPW, K), jnp.int32),
            pltpu.VMEM((K,), jnp.float32),
            pltpu.VMEM((NSLICE,), jnp.float32),
            pltpu.VMEM_SHARED((NPAD,), jnp.float32),
        ],
    )(row2d)


def _scale_body(out_ref, t_ref, m_ref, degp_ref, a_ref, sup_ref):
    deg = degp_ref[0] + degp_ref[1]
    inv = lax.rsqrt(deg)
    a_ref[0:N, :] = out_ref[...] * inv[0:N]
    a_ref[N:NPAD, :] = jnp.zeros((NPAD - N, C), jnp.float32)
    iota = lax.broadcasted_iota(jnp.int32, (N, C), 1)
    onehot = (iota == t_ref[...]).astype(jnp.float32)
    sup_sum = jnp.sum(onehot * m_ref[...] * (-out_ref[...]))
    msum = jnp.sum(m_ref[...])
    sup_ref[...] = jnp.reshape(sup_sum / jnp.maximum(msum, 1.0), (1, 1))


def _scale_call(output, t2d, m2d, degp3):
    return pl.pallas_call(
        _scale_body,
        out_shape=(
            jax.ShapeDtypeStruct((NPAD, C), jnp.float32),
            jax.ShapeDtypeStruct((1, 1), jnp.float32),
        ),
    )(output, t2d, m2d, degp3)


def _edge_body(a_hbm, rowi_hbm, coli_hbm, part_out,
               idxr, idxc, bufr, bufc, accv, semr, semc):
    c = lax.axis_index("c")
    s = lax.axis_index("s")
    w = s * NC + c
    pltpu.sync_copy(rowi_hbm.at[pl.ds(w * CPW, CPW)], idxr)
    pltpu.sync_copy(coli_hbm.at[pl.ds(w * CPW, CPW)], idxc)
    zero = jnp.zeros((L,), jnp.float32)

    def chunk(j, accs):
        cr = pltpu.async_copy(a_hbm.at[idxr.at[j]], bufr, semr)
        cc = pltpu.async_copy(a_hbm.at[idxc.at[j]], bufc, semc)
        cr.wait()
        cc.wait()

        def edge(e, accs):
            new = []
            for t in range(C // L):
                d = bufr[e, pl.ds(t * L, L)] - bufc[e, pl.ds(t * L, L)]
                new.append(accs[t] + d * d)
            return tuple(new)

        return lax.fori_loop(0, K, edge, accs)

    accs = lax.fori_loop(0, CPW, chunk, (zero,) * (C // L))
    for t in range(C // L):
        accv[pl.ds(t * L, L)] = accs[t]
    pltpu.sync_copy(accv, part_out.at[w])


def _edge_call(a, row2d, col2d):
    return pl.kernel(
        _edge_body,
        out_type=jax.ShapeDtypeStruct((NW, K), jnp.float32),
        mesh=plsc.VectorSubcoreMesh(core_axis_name="c", subcore_axis_name="s"),
        scratch_types=[
            pltpu.VMEM((CPW, K), jnp.int32),
            pltpu.VMEM((CPW, K), jnp.int32),
            pltpu.VMEM((K, C), jnp.float32),
            pltpu.VMEM((K, C), jnp.float32),
            pltpu.VMEM((C,), jnp.float32),
            pltpu.SemaphoreType.DMA,
            pltpu.SemaphoreType.DMA,
        ],
    )(a, row2d, col2d)


def _combine_body(part_ref, sup_ref, loss_ref):
    smooth = jnp.sum(part_ref[...]) / float(E * C)
    loss_ref[...] = sup_ref[...] + MU * jnp.reshape(smooth, (1, 1))


def _combine_call(parts, sup):
    return pl.pallas_call(
        _combine_body,
        out_shape=jax.ShapeDtypeStruct((1, 1), jnp.float32),
    )(parts, sup)


def kernel(output, target, train_mask, edge_index, x):
    output = output.astype(jnp.float32)
    row = edge_index[0].astype(jnp.int32)
    col = edge_index[1].astype(jnp.int32)
    npad_e = EP - E
    pad_ids = N + (jnp.arange(npad_e, dtype=jnp.int32) % (NPAD - N))
    row_p = jnp.concatenate([row, pad_ids]).reshape(NCH_PAD, K)
    col_p = jnp.concatenate([col, pad_ids]).reshape(NCH_PAD, K)

    deg_parts = _degree_call(row_p)
    degp3 = deg_parts.reshape(NC, NPAD, 1)
    t2d = target.astype(jnp.int32).reshape(N, 1)
    m2d = train_mask.astype(jnp.float32).reshape(N, 1)
    a, sup = _scale_call(output, t2d, m2d, degp3)
    parts = _edge_call(a, row_p, col_p)
    loss = _combine_call(parts, sup)
    return loss.reshape(())


# speedup vs baseline: 7.2820x; 1.5339x over previous
import jax
import jax.numpy as jnp
from jax import lax
from jax.experimental import pallas as pl
from jax.experimental.pallas import tpu as pltpu
from jax.experimental.pallas import tpu_sc as plsc

N = 10000
C = 128
E = 320000
MU = 0.01

NC, NS, L = 2, 16, 16
NW = NC * NS
K = 128
NCHUNK = -(-E // K)
_CPW_RAW = -(-NCHUNK // NW)
CPW = ((_CPW_RAW + 7) // 8) * 8
NCH_PAD = CPW * NW
EP = NCH_PAD * K
NPAD = 10240
NSLICE = NPAD // NS


def _degree_body(row_hbm, deg_out, idx_all, ones_v, slice_v, deg_sh):
    c = lax.axis_index("c")
    s = lax.axis_index("s")
    w = s * NC + c

    def zb(k, carry):
        slice_v[pl.ds(k * L, L)] = jnp.zeros((L,), jnp.float32)
        return carry

    lax.fori_loop(0, NSLICE // L, zb, 0)
    pltpu.sync_copy(slice_v, deg_sh.at[pl.ds(s * NSLICE, NSLICE)])
    for t in range(K // L):
        ones_v[pl.ds(t * L, L)] = jnp.ones((L,), jnp.float32)
    pltpu.sync_copy(row_hbm.at[pl.ds(w * CPW, CPW)], idx_all)
    plsc.subcore_barrier()

    def body(j, carry):
        pltpu.sync_copy(ones_v, deg_sh.at[idx_all.at[j]], add=True)
        return carry

    lax.fori_loop(0, CPW, body, 0)
    plsc.subcore_barrier()
    pltpu.sync_copy(deg_sh.at[pl.ds(s * NSLICE, NSLICE)], slice_v)
    pltpu.sync_copy(slice_v, deg_out.at[c, pl.ds(s * NSLICE, NSLICE)])


def _degree_call(row2d):
    return pl.kernel(
        _degree_body,
        out_type=jax.ShapeDtypeStruct((NC, NPAD), jnp.float32),
        mesh=plsc.VectorSubcoreMesh(core_axis_name="c", subcore_axis_name="s"),
        scratch_types=[
            pltpu.VMEM((CPW, K), jnp.int32),
            pltpu.VMEM((K,), jnp.float32),
            pltpu.VMEM((NSLICE,), jnp.float32),
            pltpu.VMEM_SHARED((NPAD,), jnp.float32),
        ],
    )(row2d)


def _scale_body(out_ref, t_ref, m_ref, degp_ref, a_ref, sup_ref):
    deg = degp_ref[0] + degp_ref[1]
    inv = lax.rsqrt(deg)
    a_ref[0:N, :] = out_ref[...] * inv[0:N]
    a_ref[N:NPAD, :] = jnp.zeros((NPAD - N, C), jnp.float32)
    iota = lax.broadcasted_iota(jnp.int32, (N, C), 1)
    onehot = (iota == t_ref[...]).astype(jnp.float32)
    sup_sum = jnp.sum(onehot * m_ref[...] * (-out_ref[...]))
    msum = jnp.sum(m_ref[...])
    sup_ref[...] = jnp.reshape(sup_sum / jnp.maximum(msum, 1.0), (1, 1))


def _scale_call(output, t2d, m2d, degp3):
    return pl.pallas_call(
        _scale_body,
        out_shape=(
            jax.ShapeDtypeStruct((NPAD, C), jnp.float32),
            jax.ShapeDtypeStruct((1, 1), jnp.float32),
        ),
    )(output, t2d, m2d, degp3)


def _edge_body(a_hbm, rowi_hbm, coli_hbm, part_out,
               idxr, idxc, bufr0, bufc0, bufr1, bufc1, accv,
               semr0, semc0, semr1, semc1):
    c = lax.axis_index("c")
    s = lax.axis_index("s")
    w = s * NC + c
    pltpu.sync_copy(rowi_hbm.at[pl.ds(w * CPW, CPW)], idxr)
    pltpu.sync_copy(coli_hbm.at[pl.ds(w * CPW, CPW)], idxc)
    zero = jnp.zeros((L,), jnp.float32)
    slots = ((bufr0, bufc0, semr0, semc0), (bufr1, bufc1, semr1, semc1))

    def fire(j, slot):
        br, bc, sr, sc_ = slot
        pltpu.async_copy(a_hbm.at[idxr.at[j]], br, sr)
        pltpu.async_copy(a_hbm.at[idxc.at[j]], bc, sc_)

    def drain(slot):
        br, bc, sr, sc_ = slot
        pltpu.make_async_copy(a_hbm.at[idxr.at[0]], br, sr).wait()
        pltpu.make_async_copy(a_hbm.at[idxc.at[0]], bc, sc_).wait()

    def compute(slot, accs):
        br, bc, _, _ = slot

        def edge(e, accs):
            new = []
            for t in range(C // L):
                d = br[e, pl.ds(t * L, L)] - bc[e, pl.ds(t * L, L)]
                new.append(accs[t] + d * d)
            return tuple(new)

        return lax.fori_loop(0, K, edge, accs)

    fire(0, slots[0])

    def body(j2, accs):
        j = 2 * j2
        fire(j + 1, slots[1])
        drain(slots[0])
        accs = compute(slots[0], accs)

        @pl.when(j2 < CPW // 2 - 1)
        def _():
            fire(j + 2, slots[0])

        drain(slots[1])
        return compute(slots[1], accs)

    accs = lax.fori_loop(0, CPW // 2, body, (zero,) * (C // L))
    for t in range(C // L):
        accv[pl.ds(t * L, L)] = accs[t]
    pltpu.sync_copy(accv, part_out.at[w])


def _edge_call(a, row2d, col2d):
    return pl.kernel(
        _edge_body,
        out_type=jax.ShapeDtypeStruct((NW, K), jnp.float32),
        mesh=plsc.VectorSubcoreMesh(core_axis_name="c", subcore_axis_name="s"),
        scratch_types=[
            pltpu.VMEM((CPW, K), jnp.int32),
            pltpu.VMEM((CPW, K), jnp.int32),
            pltpu.VMEM((K, C), jnp.float32),
            pltpu.VMEM((K, C), jnp.float32),
            pltpu.VMEM((K, C), jnp.float32),
            pltpu.VMEM((K, C), jnp.float32),
            pltpu.VMEM((C,), jnp.float32),
            pltpu.SemaphoreType.DMA,
            pltpu.SemaphoreType.DMA,
            pltpu.SemaphoreType.DMA,
            pltpu.SemaphoreType.DMA,
        ],
    )(a, row2d, col2d)


def _combine_body(part_ref, sup_ref, loss_ref):
    smooth = jnp.sum(part_ref[...]) / float(E * C)
    loss_ref[...] = sup_ref[...] + MU * jnp.reshape(smooth, (1, 1))


def _combine_call(parts, sup):
    return pl.pallas_call(
        _combine_body,
        out_shape=jax.ShapeDtypeStruct((1, 1), jnp.float32),
    )(parts, sup)


def kernel(output, target, train_mask, edge_index, x):
    output = output.astype(jnp.float32)
    row = edge_index[0].astype(jnp.int32)
    col = edge_index[1].astype(jnp.int32)
    npad_e = EP - E
    pad_ids = N + (jnp.arange(npad_e, dtype=jnp.int32) % (NPAD - N))
    row_p = jnp.concatenate([row, pad_ids]).reshape(NCH_PAD, K)
    col_p = jnp.concatenate([col, pad_ids]).reshape(NCH_PAD, K)

    deg_parts = _degree_call(row_p)
    degp3 = deg_parts.reshape(NC, NPAD, 1)
    t2d = target.astype(jnp.int32).reshape(N, 1)
    m2d = train_mask.astype(jnp.float32).reshape(N, 1)
    a, sup = _scale_call(output, t2d, m2d, degp3)
    parts = _edge_call(a, row_p, col_p)
    loss = _combine_call(parts, sup)
    return loss.reshape(())


# speedup vs baseline: 7.8071x; 1.0721x over previous
import jax
import jax.numpy as jnp
from jax import lax
from jax.experimental import pallas as pl
from jax.experimental.pallas import tpu as pltpu
from jax.experimental.pallas import tpu_sc as plsc

N = 10000
C = 128
E = 320000
MU = 0.01

NC, NS, L = 2, 16, 16
NW = NC * NS
K = 128
NCHUNK = -(-E // K)
_CPW_RAW = -(-NCHUNK // NW)
CPW = ((_CPW_RAW + 7) // 8) * 8
NCH_PAD = CPW * NW
EP = NCH_PAD * K
NPAD = 10240
NSLICE = NPAD // NS


def _degree_body(row_hbm, deg_out, idx_all, ones_v, slice_v, deg_sh):
    c = lax.axis_index("c")
    s = lax.axis_index("s")
    w = s * NC + c

    def zb(k, carry):
        slice_v[pl.ds(k * L, L)] = jnp.zeros((L,), jnp.float32)
        return carry

    lax.fori_loop(0, NSLICE // L, zb, 0)
    pltpu.sync_copy(slice_v, deg_sh.at[pl.ds(s * NSLICE, NSLICE)])
    for t in range(K // L):
        ones_v[pl.ds(t * L, L)] = jnp.ones((L,), jnp.float32)
    pltpu.sync_copy(row_hbm.at[pl.ds(w * CPW, CPW)], idx_all)
    plsc.subcore_barrier()

    def body(j, carry):
        pltpu.sync_copy(ones_v, deg_sh.at[idx_all.at[j]], add=True)
        return carry

    lax.fori_loop(0, CPW, body, 0)
    plsc.subcore_barrier()
    pltpu.sync_copy(deg_sh.at[pl.ds(s * NSLICE, NSLICE)], slice_v)
    pltpu.sync_copy(slice_v, deg_out.at[c, pl.ds(s * NSLICE, NSLICE)])


def _degree_call(row2d):
    return pl.kernel(
        _degree_body,
        out_type=jax.ShapeDtypeStruct((NC, NPAD), jnp.float32),
        mesh=plsc.VectorSubcoreMesh(core_axis_name="c", subcore_axis_name="s"),
        scratch_types=[
            pltpu.VMEM((CPW, K), jnp.int32),
            pltpu.VMEM((K,), jnp.float32),
            pltpu.VMEM((NSLICE,), jnp.float32),
            pltpu.VMEM_SHARED((NPAD,), jnp.float32),
        ],
    )(row2d)


def _scale_body(out_ref, t_ref, m_ref, degp_ref, a_ref, sup_ref):
    deg = degp_ref[0] + degp_ref[1]
    inv = lax.rsqrt(deg)
    a_ref[0:N, :] = (out_ref[...] * inv[0:N]).astype(jnp.bfloat16)
    a_ref[N:NPAD, :] = jnp.zeros((NPAD - N, C), jnp.bfloat16)
    iota = lax.broadcasted_iota(jnp.int32, (N, C), 1)
    onehot = (iota == t_ref[...]).astype(jnp.float32)
    sup_sum = jnp.sum(onehot * m_ref[...] * (-out_ref[...]))
    msum = jnp.sum(m_ref[...])
    sup_ref[...] = jnp.reshape(sup_sum / jnp.maximum(msum, 1.0), (1, 1))


def _scale_call(output, t2d, m2d, degp3):
    return pl.pallas_call(
        _scale_body,
        out_shape=(
            jax.ShapeDtypeStruct((NPAD, C), jnp.bfloat16),
            jax.ShapeDtypeStruct((1, 1), jnp.float32),
        ),
    )(output, t2d, m2d, degp3)


def _edge_body(a_hbm, rowi_hbm, coli_hbm, part_out,
               idxr, idxc, bufr0, bufc0, bufr1, bufc1, accv,
               semr0, semc0, semr1, semc1):
    c = lax.axis_index("c")
    s = lax.axis_index("s")
    w = s * NC + c
    pltpu.sync_copy(rowi_hbm.at[pl.ds(w * CPW, CPW)], idxr)
    pltpu.sync_copy(coli_hbm.at[pl.ds(w * CPW, CPW)], idxc)
    zero = jnp.zeros((L,), jnp.float32)
    slots = ((bufr0, bufc0, semr0, semc0), (bufr1, bufc1, semr1, semc1))

    def fire(j, slot):
        br, bc, sr, sc_ = slot
        pltpu.async_copy(a_hbm.at[idxr.at[j]], br, sr)
        pltpu.async_copy(a_hbm.at[idxc.at[j]], bc, sc_)

    def drain(slot):
        br, bc, sr, sc_ = slot
        pltpu.make_async_copy(a_hbm.at[idxr.at[0]], br, sr).wait()
        pltpu.make_async_copy(a_hbm.at[idxc.at[0]], bc, sc_).wait()

    def compute(slot, accs):
        br, bc, _, _ = slot

        mask_hi = jnp.int32(-65536)

        def edge(e, accs):
            new = list(accs)
            for t in range(C // (2 * L)):
                rw = br[e, pl.ds(t * L, L)]
                cw = bc[e, pl.ds(t * L, L)]
                r_lo = lax.bitcast_convert_type(lax.shift_left(rw, 16), jnp.float32)
                c_lo = lax.bitcast_convert_type(lax.shift_left(cw, 16), jnp.float32)
                r_hi = lax.bitcast_convert_type(lax.bitwise_and(rw, mask_hi), jnp.float32)
                c_hi = lax.bitcast_convert_type(lax.bitwise_and(cw, mask_hi), jnp.float32)
                d0 = r_lo - c_lo
                d1 = r_hi - c_hi
                new[2 * t] = new[2 * t] + d0 * d0
                new[2 * t + 1] = new[2 * t + 1] + d1 * d1
            return tuple(new)

        return lax.fori_loop(0, K, edge, accs)

    fire(0, slots[0])

    def body(j2, accs):
        j = 2 * j2
        fire(j + 1, slots[1])
        drain(slots[0])
        accs = compute(slots[0], accs)

        @pl.when(j2 < CPW // 2 - 1)
        def _():
            fire(j + 2, slots[0])

        drain(slots[1])
        return compute(slots[1], accs)

    accs = lax.fori_loop(0, CPW // 2, body, (zero,) * (C // L))
    for t in range(C // L):
        accv[pl.ds(t * L, L)] = accs[t]
    pltpu.sync_copy(accv, part_out.at[w])


def _edge_call(a, row2d, col2d):
    return pl.kernel(
        _edge_body,
        out_type=jax.ShapeDtypeStruct((NW, K), jnp.float32),
        mesh=plsc.VectorSubcoreMesh(core_axis_name="c", subcore_axis_name="s"),
        compiler_params=pltpu.CompilerParams(use_tc_tiling_on_sc=False),
        scratch_types=[
            pltpu.VMEM((CPW, K), jnp.int32),
            pltpu.VMEM((CPW, K), jnp.int32),
            pltpu.VMEM((K, C // 2), jnp.int32),
            pltpu.VMEM((K, C // 2), jnp.int32),
            pltpu.VMEM((K, C // 2), jnp.int32),
            pltpu.VMEM((K, C // 2), jnp.int32),
            pltpu.VMEM((C,), jnp.float32),
            pltpu.SemaphoreType.DMA,
            pltpu.SemaphoreType.DMA,
            pltpu.SemaphoreType.DMA,
            pltpu.SemaphoreType.DMA,
        ],
    )(a, row2d, col2d)


def _combine_body(part_ref, sup_ref, loss_ref):
    smooth = jnp.sum(part_ref[...]) / float(E * C)
    loss_ref[...] = sup_ref[...] + MU * jnp.reshape(smooth, (1, 1))


def _combine_call(parts, sup):
    return pl.pallas_call(
        _combine_body,
        out_shape=jax.ShapeDtypeStruct((1, 1), jnp.float32),
    )(parts, sup)


def kernel(output, target, train_mask, edge_index, x):
    output = output.astype(jnp.float32)
    row = edge_index[0].astype(jnp.int32)
    col = edge_index[1].astype(jnp.int32)
    npad_e = EP - E
    pad_ids = N + (jnp.arange(npad_e, dtype=jnp.int32) % (NPAD - N))
    row_p = jnp.concatenate([row, pad_ids]).reshape(NCH_PAD, K)
    col_p = jnp.concatenate([col, pad_ids]).reshape(NCH_PAD, K)

    deg_parts = _degree_call(row_p)
    degp3 = deg_parts.reshape(NC, NPAD, 1)
    t2d = target.astype(jnp.int32).reshape(N, 1)
    m2d = train_mask.astype(jnp.float32).reshape(N, 1)
    a, sup = _scale_call(output, t2d, m2d, degp3)
    a_i32 = lax.bitcast_convert_type(a.reshape(NPAD, C // 2, 2), jnp.int32)
    parts = _edge_call(a_i32, row_p, col_p)
    loss = _combine_call(parts, sup)
    return loss.reshape(())


# speedup vs baseline: 8.0723x; 1.0340x over previous
import jax
import jax.numpy as jnp
from jax import lax
from jax.experimental import pallas as pl
from jax.experimental.pallas import tpu as pltpu
from jax.experimental.pallas import tpu_sc as plsc

N = 10000
C = 128
E = 320000
MU = 0.01

NC, NS, L = 2, 16, 16
NW = NC * NS
K = 128
NCHUNK = -(-E // K)
_CPW_RAW = -(-NCHUNK // NW)
CPW = ((_CPW_RAW + 7) // 8) * 8
NCH_PAD = CPW * NW
EP = NCH_PAD * K
NPAD = 10240
NSLICE = NPAD // NS


def _degree_body(row_hbm, deg_out, idx_all, ones_v, slice_v, deg_sh):
    c = lax.axis_index("c")
    s = lax.axis_index("s")
    w = s * NC + c

    def zb(k, carry):
        slice_v[pl.ds(k * L, L)] = jnp.zeros((L,), jnp.float32)
        return carry

    lax.fori_loop(0, NSLICE // L, zb, 0)
    pltpu.sync_copy(slice_v, deg_sh.at[pl.ds(s * NSLICE, NSLICE)])
    for t in range(K // L):
        ones_v[pl.ds(t * L, L)] = jnp.ones((L,), jnp.float32)
    pltpu.sync_copy(row_hbm.at[pl.ds(w * CPW, CPW)], idx_all)
    plsc.subcore_barrier()

    def body(j, carry):
        pltpu.sync_copy(ones_v, deg_sh.at[idx_all.at[j]], add=True)
        return carry

    lax.fori_loop(0, CPW, body, 0)
    plsc.subcore_barrier()
    pltpu.sync_copy(deg_sh.at[pl.ds(s * NSLICE, NSLICE)], slice_v)
    pltpu.sync_copy(slice_v, deg_out.at[c, pl.ds(s * NSLICE, NSLICE)])


def _degree_call(row2d):
    return pl.kernel(
        _degree_body,
        out_type=jax.ShapeDtypeStruct((NC, NPAD), jnp.float32),
        mesh=plsc.VectorSubcoreMesh(core_axis_name="c", subcore_axis_name="s"),
        scratch_types=[
            pltpu.VMEM((CPW, K), jnp.int32),
            pltpu.VMEM((K,), jnp.float32),
            pltpu.VMEM((NSLICE,), jnp.float32),
            pltpu.VMEM_SHARED((NPAD,), jnp.float32),
        ],
    )(row2d)


def _scale_body(out_ref, t_ref, m_ref, degp_ref, a_ref, sup_ref):
    deg = degp_ref[0] + degp_ref[1]
    inv = lax.rsqrt(deg)
    a_ref[0:N, :] = (out_ref[...] * inv[0:N]).astype(jnp.bfloat16)
    a_ref[N:NPAD, :] = jnp.zeros((NPAD - N, C), jnp.bfloat16)
    iota = lax.broadcasted_iota(jnp.int32, (N, C), 1)
    onehot = (iota == t_ref[...]).astype(jnp.float32)
    sup_sum = jnp.sum(onehot * m_ref[...] * (-out_ref[...]))
    msum = jnp.sum(m_ref[...])
    sup_ref[...] = jnp.reshape(sup_sum / jnp.maximum(msum, 1.0), (1, 1))


def _scale_call(output, t2d, m2d, degp3):
    return pl.pallas_call(
        _scale_body,
        out_shape=(
            jax.ShapeDtypeStruct((NPAD, C), jnp.bfloat16),
            jax.ShapeDtypeStruct((1, 1), jnp.float32),
        ),
    )(output, t2d, m2d, degp3)


def _edge_body(a_hbm, rowi_hbm, coli_hbm, part_out,
               idxr, idxc, bufr0, bufc0, bufr1, bufc1, accv, a_sh,
               semr0, semc0, semr1, semc1):
    c = lax.axis_index("c")
    s = lax.axis_index("s")
    w = s * NC + c
    pltpu.sync_copy(a_hbm.at[pl.ds(s * NSLICE, NSLICE)],
                    a_sh.at[pl.ds(s * NSLICE, NSLICE)])
    pltpu.sync_copy(rowi_hbm.at[pl.ds(w * CPW, CPW)], idxr)
    pltpu.sync_copy(coli_hbm.at[pl.ds(w * CPW, CPW)], idxc)
    plsc.subcore_barrier()
    zero = jnp.zeros((L,), jnp.float32)
    slots = ((bufr0, bufc0, semr0, semc0), (bufr1, bufc1, semr1, semc1))

    def fire(j, slot):
        br, bc, sr, sc_ = slot
        pltpu.async_copy(a_sh.at[idxr.at[j]], br, sr)
        pltpu.async_copy(a_sh.at[idxc.at[j]], bc, sc_)

    def drain(slot):
        br, bc, sr, sc_ = slot
        pltpu.make_async_copy(a_hbm.at[idxr.at[0]], br, sr).wait()
        pltpu.make_async_copy(a_hbm.at[idxc.at[0]], bc, sc_).wait()

    def compute(slot, accs):
        br, bc, _, _ = slot

        mask_hi = jnp.int32(-65536)

        def edge(e, accs):
            new = list(accs)
            for t in range(C // (2 * L)):
                rw = br[e, pl.ds(t * L, L)]
                cw = bc[e, pl.ds(t * L, L)]
                r_lo = lax.bitcast_convert_type(lax.shift_left(rw, 16), jnp.float32)
                c_lo = lax.bitcast_convert_type(lax.shift_left(cw, 16), jnp.float32)
                r_hi = lax.bitcast_convert_type(lax.bitwise_and(rw, mask_hi), jnp.float32)
                c_hi = lax.bitcast_convert_type(lax.bitwise_and(cw, mask_hi), jnp.float32)
                d0 = r_lo - c_lo
                d1 = r_hi - c_hi
                new[2 * t] = new[2 * t] + d0 * d0
                new[2 * t + 1] = new[2 * t + 1] + d1 * d1
            return tuple(new)

        return lax.fori_loop(0, K, edge, accs)

    fire(0, slots[0])

    def body(j2, accs):
        j = 2 * j2
        fire(j + 1, slots[1])
        drain(slots[0])
        accs = compute(slots[0], accs)

        @pl.when(j2 < CPW // 2 - 1)
        def _():
            fire(j + 2, slots[0])

        drain(slots[1])
        return compute(slots[1], accs)

    accs = lax.fori_loop(0, CPW // 2, body, (zero,) * (C // L))
    for t in range(C // L):
        accv[pl.ds(t * L, L)] = accs[t]
    pltpu.sync_copy(accv, part_out.at[w])


def _edge_call(a, row2d, col2d):
    return pl.kernel(
        _edge_body,
        out_type=jax.ShapeDtypeStruct((NW, K), jnp.float32),
        mesh=plsc.VectorSubcoreMesh(core_axis_name="c", subcore_axis_name="s"),
        compiler_params=pltpu.CompilerParams(use_tc_tiling_on_sc=False),
        scratch_types=[
            pltpu.VMEM((CPW, K), jnp.int32),
            pltpu.VMEM((CPW, K), jnp.int32),
            pltpu.VMEM((K, C // 2), jnp.int32),
            pltpu.VMEM((K, C // 2), jnp.int32),
            pltpu.VMEM((K, C // 2), jnp.int32),
            pltpu.VMEM((K, C // 2), jnp.int32),
            pltpu.VMEM((C,), jnp.float32),
            pltpu.VMEM_SHARED((NPAD, C // 2), jnp.int32),
            pltpu.SemaphoreType.DMA,
            pltpu.SemaphoreType.DMA,
            pltpu.SemaphoreType.DMA,
            pltpu.SemaphoreType.DMA,
        ],
    )(a, row2d, col2d)


def _combine_body(part_ref, sup_ref, loss_ref):
    smooth = jnp.sum(part_ref[...]) / float(E * C)
    loss_ref[...] = sup_ref[...] + MU * jnp.reshape(smooth, (1, 1))


def _combine_call(parts, sup):
    return pl.pallas_call(
        _combine_body,
        out_shape=jax.ShapeDtypeStruct((1, 1), jnp.float32),
    )(parts, sup)


def kernel(output, target, train_mask, edge_index, x):
    output = output.astype(jnp.float32)
    row = edge_index[0].astype(jnp.int32)
    col = edge_index[1].astype(jnp.int32)
    npad_e = EP - E
    pad_ids = N + (jnp.arange(npad_e, dtype=jnp.int32) % (NPAD - N))
    row_p = jnp.concatenate([row, pad_ids]).reshape(NCH_PAD, K)
    col_p = jnp.concatenate([col, pad_ids]).reshape(NCH_PAD, K)

    deg_parts = _degree_call(row_p)
    degp3 = deg_parts.reshape(NC, NPAD, 1)
    t2d = target.astype(jnp.int32).reshape(N, 1)
    m2d = train_mask.astype(jnp.float32).reshape(N, 1)
    a, sup = _scale_call(output, t2d, m2d, degp3)
    a_i32 = lax.bitcast_convert_type(a.reshape(NPAD, C // 2, 2), jnp.int32)
    parts = _edge_call(a_i32, row_p, col_p)
    loss = _combine_call(parts, sup)
    return loss.reshape(())


# speedup vs baseline: 8.8650x; 1.0982x over previous
import jax
import jax.numpy as jnp
from jax import lax
from jax.experimental import pallas as pl
from jax.experimental.pallas import tpu as pltpu
from jax.experimental.pallas import tpu_sc as plsc

N = 10000
C = 128
E = 320000
MU = 0.01

NC, NS, L = 2, 16, 16
NW = NC * NS
K = 128
NCHUNK = -(-E // K)
_CPW_RAW = -(-NCHUNK // NW)
CPW = ((_CPW_RAW + 7) // 8) * 8
NCH_PAD = CPW * NW
EP = NCH_PAD * K
NPAD = 10240
NSLICE = NPAD // NS


def _degree_body(row_hbm, deg_out, idx_all, ones_v, slice_v, deg_sh):
    c = lax.axis_index("c")
    s = lax.axis_index("s")
    w = s * NC + c

    def zb(k, carry):
        slice_v[pl.ds(k * L, L)] = jnp.zeros((L,), jnp.float32)
        return carry

    lax.fori_loop(0, NSLICE // L, zb, 0)
    pltpu.sync_copy(slice_v, deg_sh.at[pl.ds(s * NSLICE, NSLICE)])
    for t in range(K // L):
        ones_v[pl.ds(t * L, L)] = jnp.ones((L,), jnp.float32)
    pltpu.sync_copy(row_hbm.at[pl.ds(w * CPW, CPW)], idx_all)
    plsc.subcore_barrier()

    def body(j, carry):
        pltpu.sync_copy(ones_v, deg_sh.at[idx_all.at[j]], add=True)
        return carry

    lax.fori_loop(0, CPW, body, 0)
    plsc.subcore_barrier()
    pltpu.sync_copy(deg_sh.at[pl.ds(s * NSLICE, NSLICE)], slice_v)
    pltpu.sync_copy(slice_v, deg_out.at[c, pl.ds(s * NSLICE, NSLICE)])


def _degree_call(row2d):
    return pl.kernel(
        _degree_body,
        out_type=jax.ShapeDtypeStruct((NC, NPAD), jnp.float32),
        mesh=plsc.VectorSubcoreMesh(core_axis_name="c", subcore_axis_name="s"),
        scratch_types=[
            pltpu.VMEM((CPW, K), jnp.int32),
            pltpu.VMEM((K,), jnp.float32),
            pltpu.VMEM((NSLICE,), jnp.float32),
            pltpu.VMEM_SHARED((NPAD,), jnp.float32),
        ],
    )(row2d)


def _scale_body(out_ref, t_ref, m_ref, degp_ref, a_ref, sup_ref):
    deg = degp_ref[0] + degp_ref[1]
    inv = lax.rsqrt(deg)
    a_ref[0:N, :] = (out_ref[...] * inv[0:N]).astype(jnp.bfloat16)
    a_ref[N:NPAD, :] = jnp.zeros((NPAD - N, C), jnp.bfloat16)
    iota = lax.broadcasted_iota(jnp.int32, (N, C), 1)
    onehot = (iota == t_ref[...]).astype(jnp.float32)
    sup_sum = jnp.sum(onehot * m_ref[...] * (-out_ref[...]))
    msum = jnp.sum(m_ref[...])
    sup_ref[...] = jnp.reshape(sup_sum / jnp.maximum(msum, 1.0), (1, 1))


def _scale_call(output, t2d, m2d, degp3):
    return pl.pallas_call(
        _scale_body,
        out_shape=(
            jax.ShapeDtypeStruct((NPAD, C), jnp.bfloat16),
            jax.ShapeDtypeStruct((1, 1), jnp.float32),
        ),
    )(output, t2d, m2d, degp3)


def _edge_body(a_hbm, rowi_hbm, coli_hbm, part_out,
               idxr, idxc, bufr0, bufc0, bufr1, bufc1, accv, a_sh,
               semr0, semc0, semr1, semc1):
    c = lax.axis_index("c")
    s = lax.axis_index("s")
    w = s * NC + c
    pltpu.sync_copy(a_hbm.at[pl.ds(s * NSLICE, NSLICE)],
                    a_sh.at[pl.ds(s * NSLICE, NSLICE)])
    pltpu.sync_copy(rowi_hbm.at[pl.ds(w * CPW, CPW)], idxr)
    pltpu.sync_copy(coli_hbm.at[pl.ds(w * CPW, CPW)], idxc)
    plsc.subcore_barrier()
    zero = jnp.zeros((L,), jnp.float32)
    slots = ((bufr0, bufc0, semr0, semc0), (bufr1, bufc1, semr1, semc1))

    def fire(j, slot):
        br, bc, sr, sc_ = slot
        pltpu.async_copy(a_sh.at[idxr.at[j]], br, sr)
        pltpu.async_copy(a_sh.at[idxc.at[j]], bc, sc_)

    def drain(slot):
        br, bc, sr, sc_ = slot
        pltpu.make_async_copy(a_hbm.at[idxr.at[0]], br, sr).wait()
        pltpu.make_async_copy(a_hbm.at[idxc.at[0]], bc, sc_).wait()

    def compute(slot, accs):
        br, bc, _, _ = slot

        def one_edge(e, new):
            for t in range(C // (2 * L)):
                rw = br[e, pl.ds(t * L, L)]
                cw = bc[e, pl.ds(t * L, L)]
                r_lo = lax.bitcast_convert_type(lax.shift_left(rw, 16), jnp.float32)
                c_lo = lax.bitcast_convert_type(lax.shift_left(cw, 16), jnp.float32)
                r_hi = lax.bitcast_convert_type(rw, jnp.float32)
                c_hi = lax.bitcast_convert_type(cw, jnp.float32)
                d0 = r_lo - c_lo
                d1 = r_hi - c_hi
                new[2 * t] = new[2 * t] + d0 * d0
                new[2 * t + 1] = new[2 * t + 1] + d1 * d1
            return new

        def edge2(e2, accs):
            new = list(accs)
            new = one_edge(2 * e2, new)
            new = one_edge(2 * e2 + 1, new)
            return tuple(new)

        return lax.fori_loop(0, K // 2, edge2, accs)

    fire(0, slots[0])

    def body(j2, accs):
        j = 2 * j2
        fire(j + 1, slots[1])
        drain(slots[0])
        accs = compute(slots[0], accs)

        @pl.when(j2 < CPW // 2 - 1)
        def _():
            fire(j + 2, slots[0])

        drain(slots[1])
        return compute(slots[1], accs)

    accs = lax.fori_loop(0, CPW // 2, body, (zero,) * (C // L))
    for t in range(C // L):
        accv[pl.ds(t * L, L)] = accs[t]
    pltpu.sync_copy(accv, part_out.at[w])


def _edge_call(a, row2d, col2d):
    return pl.kernel(
        _edge_body,
        out_type=jax.ShapeDtypeStruct((NW, K), jnp.float32),
        mesh=plsc.VectorSubcoreMesh(core_axis_name="c", subcore_axis_name="s"),
        compiler_params=pltpu.CompilerParams(use_tc_tiling_on_sc=False),
        scratch_types=[
            pltpu.VMEM((CPW, K), jnp.int32),
            pltpu.VMEM((CPW, K), jnp.int32),
            pltpu.VMEM((K, C // 2), jnp.int32),
            pltpu.VMEM((K, C // 2), jnp.int32),
            pltpu.VMEM((K, C // 2), jnp.int32),
            pltpu.VMEM((K, C // 2), jnp.int32),
            pltpu.VMEM((C,), jnp.float32),
            pltpu.VMEM_SHARED((NPAD, C // 2), jnp.int32),
            pltpu.SemaphoreType.DMA,
            pltpu.SemaphoreType.DMA,
            pltpu.SemaphoreType.DMA,
            pltpu.SemaphoreType.DMA,
        ],
    )(a, row2d, col2d)


def _combine_body(part_ref, sup_ref, loss_ref):
    smooth = jnp.sum(part_ref[...]) / float(E * C)
    loss_ref[...] = sup_ref[...] + MU * jnp.reshape(smooth, (1, 1))


def _combine_call(parts, sup):
    return pl.pallas_call(
        _combine_body,
        out_shape=jax.ShapeDtypeStruct((1, 1), jnp.float32),
    )(parts, sup)


def kernel(output, target, train_mask, edge_index, x):
    output = output.astype(jnp.float32)
    row = edge_index[0].astype(jnp.int32)
    col = edge_index[1].astype(jnp.int32)
    npad_e = EP - E
    pad_ids = N + (jnp.arange(npad_e, dtype=jnp.int32) % (NPAD - N))
    row_p = jnp.concatenate([row, pad_ids]).reshape(NCH_PAD, K)
    col_p = jnp.concatenate([col, pad_ids]).reshape(NCH_PAD, K)

    deg_parts = _degree_call(row_p)
    degp3 = deg_parts.reshape(NC, NPAD, 1)
    t2d = target.astype(jnp.int32).reshape(N, 1)
    m2d = train_mask.astype(jnp.float32).reshape(N, 1)
    a, sup = _scale_call(output, t2d, m2d, degp3)
    a_i32 = lax.bitcast_convert_type(a.reshape(NPAD, C // 2, 2), jnp.int32)
    parts = _edge_call(a_i32, row_p, col_p)
    loss = _combine_call(parts, sup)
    return loss.reshape(())
